# Initial kernel scaffold; baseline (speedup 1.0000x reference)
import jax
import jax.numpy as jnp
from jax.experimental import pallas as pl


def kernel(u, grid, inc):
    raise NotImplementedError("write your pallas kernel here")


# speedup vs baseline: 141.8634x; 141.8634x over previous
import functools

import jax
import jax.numpy as jnp
from jax import lax
from jax.experimental import pallas as pl
from jax.experimental.pallas import tpu as pltpu
from jax.experimental.pallas import tpu_sc as plsc


def _log_table_tc(inc, ninc):
    def body(inc_ref, out_ref):
        out_ref[...] = jnp.log(inc_ref[...] * jnp.float32(ninc))

    return pl.pallas_call(
        body,
        out_shape=jax.ShapeDtypeStruct(inc.shape, inc.dtype),
    )(inc)


def _make_sc_kernel(batch, dim, ninc, n_workers, chunk):
    spw = batch // n_workers
    n_chunks = spw // chunk
    groups = chunk // 16
    mesh = plsc.VectorSubcoreMesh(core_axis_name="c", subcore_axis_name="s")
    nc = mesh.num_cores

    grid_words = dim * (ninc + 1)
    tab_words = dim * ninc

    @functools.partial(
        pl.kernel,
        mesh=mesh,
        out_type=(
            jax.ShapeDtypeStruct((batch * dim,), jnp.float32),
            jax.ShapeDtypeStruct((batch,), jnp.float32),
        ),
        scratch_types=[
            pltpu.VMEM((grid_words,), jnp.float32),
            pltpu.VMEM((tab_words,), jnp.float32),
            pltpu.VMEM((tab_words,), jnp.float32),
            pltpu.VMEM((chunk * dim,), jnp.float32),
            pltpu.VMEM((chunk * dim,), jnp.float32),
            pltpu.VMEM((chunk,), jnp.float32),
        ],
        compiler_params=pltpu.CompilerParams(needs_layout_passes=False),
    )
    def k(u_hbm, grid_hbm, inc_hbm, log_hbm, x_hbm, ld_hbm,
          grid_v, inc_v, log_v, u_v, x_v, ld_v):
        wid = lax.axis_index("s") * nc + lax.axis_index("c")
        pltpu.sync_copy(grid_hbm, grid_v)
        pltpu.sync_copy(inc_hbm, inc_v)
        pltpu.sync_copy(log_hbm, log_v)
        base = wid * spw
        iota8 = lax.iota(jnp.int32, 16) * 8

        def do_chunk(off):
            pltpu.sync_copy(u_hbm.at[pl.ds(off * dim, chunk * dim)], u_v)

            def grp(g, carry):
                rows8 = iota8 + g * (16 * dim)
                acc = jnp.zeros((16,), jnp.float32)
                for d in range(dim):
                    idx_u = rows8 + d
                    u_d = plsc.load_gather(u_v, [idx_u])
                    uni = u_d * jnp.float32(ninc)
                    iui = uni.astype(jnp.int32)
                    iui = jnp.minimum(iui, ninc - 1)
                    iui = jnp.maximum(iui, 0)
                    du = uni - iui.astype(jnp.float32)
                    g0 = plsc.load_gather(grid_v, [iui + d * (ninc + 1)])
                    ic = plsc.load_gather(inc_v, [iui + d * ninc])
                    lg = plsc.load_gather(log_v, [iui + d * ninc])
                    plsc.store_scatter(x_v, [idx_u], g0 + ic * du)
                    acc = acc + lg
                ld_v[pl.ds(g * 16, 16)] = acc
                return carry

            lax.fori_loop(0, groups, grp, 0)
            pltpu.sync_copy(x_v, x_hbm.at[pl.ds(off * dim, chunk * dim)])
            pltpu.sync_copy(ld_v, ld_hbm.at[pl.ds(off, chunk)])

        for ci in range(n_chunks):
            do_chunk(base + ci * chunk)

    return k


def kernel(u, grid, inc):
    batch, dim = u.shape
    ninc = inc.shape[1]
    log_inc = _log_table_tc(inc, ninc)
    info = plsc.get_sparse_core_info()
    n_workers = info.num_cores * info.num_subcores
    sc = _make_sc_kernel(batch, dim, ninc, n_workers, chunk=2048)
    x_flat, log_detJ = sc(
        u.reshape(-1), grid.reshape(-1), inc.reshape(-1), log_inc.reshape(-1)
    )
    return x_flat.reshape(batch, dim), log_detJ
